# Initial kernel scaffold; baseline (speedup 1.0000x reference)
import jax
import jax.numpy as jnp
from jax.experimental import pallas as pl


def kernel(x_nhwc, g0, b0, w1, g1, b1, w2, g2, b2, wf1, bf1, wf2, bf2):
    raise NotImplementedError("write your pallas kernel here")


# speedup vs baseline: 1.7826x; 1.7826x over previous
import functools

import jax
import jax.numpy as jnp
from jax.experimental import pallas as pl
from jax.experimental.pallas import tpu as pltpu

_EPS = 1e-5
_VMEM_LIMIT = 32 * 1024 * 1024


def _sigmoid(t):
    return 1.0 / (1.0 + jnp.exp(-t))


def _silu(t):
    return t * _sigmoid(t)


def _stats_kernel(x_ref, o_ref):
    x = x_ref[...].astype(jnp.float32)
    s = jnp.sum(x, axis=0, keepdims=True)
    sq = jnp.sum(x * x, axis=0, keepdims=True)
    o_ref[...] = jnp.concatenate([s, sq], axis=0)[None]


def _conv_kernel(x_ref, scale_ref, shift_ref, w_ref, y_ref, st_ref, pad_ref,
                 *, h, w, c_in, c_out, apply_silu):
    a = x_ref[...].astype(jnp.float32).reshape(h * w, c_in)
    a = a * scale_ref[...] + shift_ref[...]
    if apply_silu:
        a = _silu(a)

    pad_ref[0:1, :, :] = jnp.zeros((1, w + 2, c_in), jnp.bfloat16)
    pad_ref[h + 1:h + 2, :, :] = jnp.zeros((1, w + 2, c_in), jnp.bfloat16)
    pad_ref[:, 0:1, :] = jnp.zeros((h + 2, 1, c_in), jnp.bfloat16)
    pad_ref[:, w + 1:w + 2, :] = jnp.zeros((h + 2, 1, c_in), jnp.bfloat16)
    pad_ref[1:h + 1, 1:w + 1, :] = a.astype(jnp.bfloat16).reshape(h, w, c_in)
    padded = pad_ref[...]

    cols = [padded[kh:kh + h, kw:kw + w, :].reshape(h * w, c_in)
            for kh in range(3) for kw in range(3)]
    patches = jnp.concatenate(cols, axis=1)
    y = jnp.dot(patches, w_ref[...], preferred_element_type=jnp.float32)

    s = jnp.sum(y, axis=0, keepdims=True)
    sq = jnp.sum(y * y, axis=0, keepdims=True)
    st_ref[...] = jnp.concatenate([s, sq], axis=0).reshape(1, 2, c_out)
    y_ref[...] = y.reshape(1, h, w, c_out).astype(y_ref.dtype)


def _bn_se_residual_kernel(y_ref, x_ref, scale_ref, shift_ref,
                           wf1_ref, bf1_ref, wf2_ref, bf2_ref, o_ref,
                           *, h, w, c):
    z = y_ref[...].astype(jnp.float32).reshape(h * w, c)
    z = z * scale_ref[...] + shift_ref[...]

    pooled = jnp.sum(z, axis=0, keepdims=True) * (1.0 / (h * w))
    g = _silu(jnp.dot(pooled, wf1_ref[...],
                      preferred_element_type=jnp.float32) + bf1_ref[...])
    g = _sigmoid(jnp.dot(g, wf2_ref[...],
                         preferred_element_type=jnp.float32) + bf2_ref[...])

    x = x_ref[...].astype(jnp.float32).reshape(h * w, c)
    out = _silu(z * g + x)
    o_ref[...] = out.reshape(1, h, w, c).astype(o_ref.dtype)


def _fold_bn(s, sq, count, gamma, beta):
    mean = s / count
    var = jnp.maximum(sq / count - mean * mean, 0.0)
    scale = gamma * jax.lax.rsqrt(var + _EPS)
    shift = beta - mean * scale
    return scale, shift


def _channel_stats(x2d, c):
    rows = x2d.shape[0]
    n_chunks = 16
    while rows % n_chunks:
        n_chunks //= 2
    rt = rows // n_chunks
    part = pl.pallas_call(
        _stats_kernel,
        grid=(n_chunks,),
        in_specs=[pl.BlockSpec((rt, c), lambda i: (i, 0))],
        out_specs=pl.BlockSpec((1, 2, c), lambda i: (i, 0, 0)),
        out_shape=jax.ShapeDtypeStruct((n_chunks, 2, c), jnp.float32),
        compiler_params=pltpu.CompilerParams(
            dimension_semantics=("parallel",),
            vmem_limit_bytes=_VMEM_LIMIT),
    )(x2d)
    return jnp.sum(part, axis=0)


def _affine_conv3x3(x, scale, shift, wcol, *, apply_silu):
    n, h, w, c_in = x.shape
    c_out = wcol.shape[1]
    kfn = functools.partial(_conv_kernel, h=h, w=w,
                            c_in=c_in, c_out=c_out, apply_silu=apply_silu)
    return pl.pallas_call(
        kfn,
        grid=(n,),
        in_specs=[
            pl.BlockSpec((1, h, w, c_in), lambda i: (i, 0, 0, 0)),
            pl.BlockSpec((1, c_in), lambda i: (0, 0)),
            pl.BlockSpec((1, c_in), lambda i: (0, 0)),
            pl.BlockSpec((9 * c_in, c_out), lambda i: (0, 0)),
        ],
        out_specs=(
            pl.BlockSpec((1, h, w, c_out), lambda i: (i, 0, 0, 0)),
            pl.BlockSpec((1, 2, c_out), lambda i: (i, 0, 0)),
        ),
        out_shape=(
            jax.ShapeDtypeStruct((n, h, w, c_out), jnp.bfloat16),
            jax.ShapeDtypeStruct((n, 2, c_out), jnp.float32),
        ),
        scratch_shapes=[pltpu.VMEM((h + 2, w + 2, c_in), jnp.bfloat16)],
        compiler_params=pltpu.CompilerParams(
            dimension_semantics=("parallel",),
            vmem_limit_bytes=_VMEM_LIMIT),
    )(x, scale, shift, wcol)


def _bn_se_residual(y, x, scale, shift, wf1, bf1, wf2, bf2):
    n, h, w, c = y.shape
    c_red = wf1.shape[1]
    kfn = functools.partial(_bn_se_residual_kernel, h=h, w=w, c=c)
    return pl.pallas_call(
        kfn,
        grid=(n,),
        in_specs=[
            pl.BlockSpec((1, h, w, c), lambda i: (i, 0, 0, 0)),
            pl.BlockSpec((1, h, w, c), lambda i: (i, 0, 0, 0)),
            pl.BlockSpec((1, c), lambda i: (0, 0)),
            pl.BlockSpec((1, c), lambda i: (0, 0)),
            pl.BlockSpec((c, c_red), lambda i: (0, 0)),
            pl.BlockSpec((1, c_red), lambda i: (0, 0)),
            pl.BlockSpec((c_red, c), lambda i: (0, 0)),
            pl.BlockSpec((1, c), lambda i: (0, 0)),
        ],
        out_specs=pl.BlockSpec((1, h, w, c), lambda i: (i, 0, 0, 0)),
        out_shape=jax.ShapeDtypeStruct((n, h, w, c), x.dtype),
        compiler_params=pltpu.CompilerParams(
            dimension_semantics=("parallel",),
            vmem_limit_bytes=_VMEM_LIMIT),
    )(y, x, scale, shift, wf1, bf1, wf2, bf2)


def kernel(x_nhwc, g0, b0, w1, g1, b1, w2, g2, b2, wf1, bf1, wf2, bf2):
    n, h, w, c = x_nhwc.shape
    count = float(n * h * w)

    w1col = w1.reshape(9 * c, c).astype(jnp.bfloat16)
    w2col = w2.reshape(9 * c, c).astype(jnp.bfloat16)

    st_x = _channel_stats(x_nhwc.reshape(n * h * w, c), c)
    scale0, shift0 = _fold_bn(st_x[0:1], st_x[1:2], count, g0, b0)

    y1, p1 = _affine_conv3x3(x_nhwc, scale0, shift0, w1col, apply_silu=False)
    s1 = jnp.sum(p1, axis=0)
    scale1, shift1 = _fold_bn(s1[0:1], s1[1:2], count, g1, b1)

    y2, p2 = _affine_conv3x3(y1, scale1, shift1, w2col, apply_silu=True)
    s2 = jnp.sum(p2, axis=0)
    scale2, shift2 = _fold_bn(s2[0:1], s2[1:2], count, g2, b2)

    return _bn_se_residual(y2, x_nhwc, scale2, shift2, wf1, bf1, wf2, bf2)


# speedup vs baseline: 2.5363x; 1.4228x over previous
import functools

import jax
import jax.numpy as jnp
from jax.experimental import pallas as pl
from jax.experimental.pallas import tpu as pltpu

_EPS = 1e-5
_VMEM_LIMIT = 32 * 1024 * 1024
_IMGS_PER_STEP = 4


def _sigmoid(t):
    return 1.0 / (1.0 + jnp.exp(-t))


def _silu(t):
    return t * _sigmoid(t)


def _stats_kernel(x_ref, o_ref):
    x = x_ref[...].astype(jnp.float32)
    s = jnp.sum(x, axis=0, keepdims=True)
    sq = jnp.sum(x * x, axis=0, keepdims=True)
    o_ref[...] = jnp.concatenate([s, sq], axis=0)[None]


def _conv_kernel(x_ref, scale_ref, shift_ref, w_ref, y_ref, st_ref,
                 bl_ref, bm_ref, br_ref, *, b, h, w, c_in, c_out, apply_silu):
    zrow = jnp.zeros((1, w, c_in), jnp.bfloat16)
    s_acc = jnp.zeros((1, c_out), jnp.float32)
    sq_acc = jnp.zeros((1, c_out), jnp.float32)

    for k in range(b):
        a = x_ref[k].astype(jnp.float32).reshape(h * w, c_in)
        a = a * scale_ref[...] + shift_ref[...]
        if apply_silu:
            a = _silu(a)
        ab = a.astype(jnp.bfloat16)

        bm_ref[0:1] = zrow
        bm_ref[h + 1:h + 2] = zrow
        bm_ref[1:h + 1] = ab.reshape(h, w, c_in)

        bl_ref[0:1] = zrow
        bl_ref[h + 1:h + 2] = zrow
        bl_ref[1:h + 1] = jnp.roll(ab, 1, axis=0).reshape(h, w, c_in)
        bl_ref[1:h + 1, 0:1, :] = jnp.zeros((h, 1, c_in), jnp.bfloat16)

        br_ref[0:1] = zrow
        br_ref[h + 1:h + 2] = zrow
        br_ref[1:h + 1] = jnp.roll(ab, -1, axis=0).reshape(h, w, c_in)
        br_ref[1:h + 1, w - 1:w, :] = jnp.zeros((h, 1, c_in), jnp.bfloat16)

        y = jnp.zeros((h * w, c_out), jnp.float32)
        for kh in range(3):
            for kw, buf in ((0, bl_ref), (1, bm_ref), (2, br_ref)):
                tap = buf[kh:kh + h].reshape(h * w, c_in)
                wt = w_ref[(kh * 3 + kw) * c_in:(kh * 3 + kw + 1) * c_in, :]
                y = y + jnp.dot(tap, wt, preferred_element_type=jnp.float32)

        y_ref[k] = y.reshape(h, w, c_out).astype(y_ref.dtype)
        s_acc = s_acc + jnp.sum(y, axis=0, keepdims=True)
        sq_acc = sq_acc + jnp.sum(y * y, axis=0, keepdims=True)

    st_ref[...] = jnp.concatenate([s_acc, sq_acc], axis=0)[None]


def _bn_se_residual_kernel(y_ref, x_ref, scale_ref, shift_ref,
                           wf1_ref, bf1_ref, wf2_ref, bf2_ref, o_ref,
                           *, b, h, w, c):
    for k in range(b):
        z = y_ref[k].astype(jnp.float32).reshape(h * w, c)
        z = z * scale_ref[...] + shift_ref[...]

        pooled = jnp.sum(z, axis=0, keepdims=True) * (1.0 / (h * w))
        g = _silu(jnp.dot(pooled, wf1_ref[...],
                          preferred_element_type=jnp.float32) + bf1_ref[...])
        g = _sigmoid(jnp.dot(g, wf2_ref[...],
                             preferred_element_type=jnp.float32) + bf2_ref[...])

        x = x_ref[k].astype(jnp.float32).reshape(h * w, c)
        out = _silu(z * g + x)
        o_ref[k] = out.reshape(h, w, c).astype(o_ref.dtype)


def _fold_bn(s, sq, count, gamma, beta):
    mean = s / count
    var = jnp.maximum(sq / count - mean * mean, 0.0)
    scale = gamma * jax.lax.rsqrt(var + _EPS)
    shift = beta - mean * scale
    return scale, shift


def _channel_stats(x2d, c):
    rows = x2d.shape[0]
    n_chunks = 16
    while rows % n_chunks:
        n_chunks //= 2
    rt = rows // n_chunks
    part = pl.pallas_call(
        _stats_kernel,
        grid=(n_chunks,),
        in_specs=[pl.BlockSpec((rt, c), lambda i: (i, 0))],
        out_specs=pl.BlockSpec((1, 2, c), lambda i: (i, 0, 0)),
        out_shape=jax.ShapeDtypeStruct((n_chunks, 2, c), jnp.float32),
        compiler_params=pltpu.CompilerParams(
            dimension_semantics=("parallel",),
            vmem_limit_bytes=_VMEM_LIMIT),
    )(x2d)
    return jnp.sum(part, axis=0)


def _affine_conv3x3(x, scale, shift, wcol, *, apply_silu):
    n, h, w, c_in = x.shape
    c_out = wcol.shape[1]
    b = _IMGS_PER_STEP
    while n % b:
        b //= 2
    kfn = functools.partial(_conv_kernel, b=b, h=h, w=w,
                            c_in=c_in, c_out=c_out, apply_silu=apply_silu)
    return pl.pallas_call(
        kfn,
        grid=(n // b,),
        in_specs=[
            pl.BlockSpec((b, h, w, c_in), lambda i: (i, 0, 0, 0)),
            pl.BlockSpec((1, c_in), lambda i: (0, 0)),
            pl.BlockSpec((1, c_in), lambda i: (0, 0)),
            pl.BlockSpec((9 * c_in, c_out), lambda i: (0, 0)),
        ],
        out_specs=(
            pl.BlockSpec((b, h, w, c_out), lambda i: (i, 0, 0, 0)),
            pl.BlockSpec((1, 2, c_out), lambda i: (i, 0, 0)),
        ),
        out_shape=(
            jax.ShapeDtypeStruct((n, h, w, c_out), jnp.bfloat16),
            jax.ShapeDtypeStruct((n // b, 2, c_out), jnp.float32),
        ),
        scratch_shapes=[pltpu.VMEM((h + 2, w, c_in), jnp.bfloat16),
                        pltpu.VMEM((h + 2, w, c_in), jnp.bfloat16),
                        pltpu.VMEM((h + 2, w, c_in), jnp.bfloat16)],
        compiler_params=pltpu.CompilerParams(
            dimension_semantics=("parallel",),
            vmem_limit_bytes=_VMEM_LIMIT),
    )(x, scale, shift, wcol)


def _bn_se_residual(y, x, scale, shift, wf1, bf1, wf2, bf2):
    n, h, w, c = y.shape
    c_red = wf1.shape[1]
    b = _IMGS_PER_STEP
    while n % b:
        b //= 2
    kfn = functools.partial(_bn_se_residual_kernel, b=b, h=h, w=w, c=c)
    return pl.pallas_call(
        kfn,
        grid=(n // b,),
        in_specs=[
            pl.BlockSpec((b, h, w, c), lambda i: (i, 0, 0, 0)),
            pl.BlockSpec((b, h, w, c), lambda i: (i, 0, 0, 0)),
            pl.BlockSpec((1, c), lambda i: (0, 0)),
            pl.BlockSpec((1, c), lambda i: (0, 0)),
            pl.BlockSpec((c, c_red), lambda i: (0, 0)),
            pl.BlockSpec((1, c_red), lambda i: (0, 0)),
            pl.BlockSpec((c_red, c), lambda i: (0, 0)),
            pl.BlockSpec((1, c), lambda i: (0, 0)),
        ],
        out_specs=pl.BlockSpec((b, h, w, c), lambda i: (i, 0, 0, 0)),
        out_shape=jax.ShapeDtypeStruct((n, h, w, c), x.dtype),
        compiler_params=pltpu.CompilerParams(
            dimension_semantics=("parallel",),
            vmem_limit_bytes=_VMEM_LIMIT),
    )(y, x, scale, shift, wf1, bf1, wf2, bf2)


def kernel(x_nhwc, g0, b0, w1, g1, b1, w2, g2, b2, wf1, bf1, wf2, bf2):
    n, h, w, c = x_nhwc.shape
    count = float(n * h * w)

    w1col = w1.reshape(9 * c, c).astype(jnp.bfloat16)
    w2col = w2.reshape(9 * c, c).astype(jnp.bfloat16)

    st_x = _channel_stats(x_nhwc.reshape(n * h * w, c), c)
    scale0, shift0 = _fold_bn(st_x[0:1], st_x[1:2], count, g0, b0)

    y1, p1 = _affine_conv3x3(x_nhwc, scale0, shift0, w1col, apply_silu=False)
    s1 = jnp.sum(p1, axis=0)
    scale1, shift1 = _fold_bn(s1[0:1], s1[1:2], count, g1, b1)

    y2, p2 = _affine_conv3x3(y1, scale1, shift1, w2col, apply_silu=True)
    s2 = jnp.sum(p2, axis=0)
    scale2, shift2 = _fold_bn(s2[0:1], s2[1:2], count, g2, b2)

    return _bn_se_residual(y2, x_nhwc, scale2, shift2, wf1, bf1, wf2, bf2)
